# Initial kernel scaffold; baseline (speedup 1.0000x reference)
import functools

import jax
import jax.numpy as jnp
from jax import lax
from jax.experimental import pallas as pl
from jax.experimental.pallas import tpu as pltpu
from jax.experimental.pallas import tpu_sc as plsc

NUM_ROWS = 1000
D = 64
B = 16384 * 200

NC, NS = 2, 16
NW = NC * NS
KS = 128
NK = 4
CHUNK = NK * KS
ROWS_PER_W = B // NW // KS
NITER = ROWS_PER_W // NK


def _emb_body(w_hbm, idx_hbm, out_hbm, idx_v, rows_v, gsem):
    wid = lax.axis_index("s") * NC + lax.axis_index("c")
    row0 = wid * ROWS_PER_W

    def step(g, carry):
        r = row0 + g * NK
        pltpu.sync_copy(idx_hbm.at[pl.ds(r, NK)], idx_v)
        cps = [
            pltpu.async_copy(
                w_hbm.at[idx_v.at[j]],
                rows_v.at[pl.ds(j * KS, KS)],
                gsem,
            )
            for j in range(NK)
        ]
        for c in cps:
            c.wait()
        pltpu.sync_copy(rows_v, out_hbm.at[pl.ds(r * KS, CHUNK)])
        return carry

    lax.fori_loop(0, NITER, step, 0)


_emb = functools.partial(
    pl.kernel,
    out_type=jax.ShapeDtypeStruct((B, D), jnp.float32),
    mesh=plsc.VectorSubcoreMesh(core_axis_name="c", subcore_axis_name="s"),
    scratch_types=[
        pltpu.VMEM((NK, KS), jnp.int32),
        pltpu.VMEM((CHUNK, D), jnp.float32),
        pltpu.SemaphoreType.DMA,
    ],
)(_emb_body)


def kernel(joint_indices, weight):
    idx2d = joint_indices.reshape(-1, KS).astype(jnp.int32)
    out = _emb(weight, idx2d)
    return out.reshape(joint_indices.shape + (D,))


# speedup vs baseline: 4.1480x; 4.1480x over previous
import functools

import jax
import jax.numpy as jnp
from jax import lax
from jax.experimental import pallas as pl
from jax.experimental.pallas import tpu as pltpu
from jax.experimental.pallas import tpu_sc as plsc

NUM_ROWS = 1000
D = 64
B = 16384 * 200

NC, NS = 2, 16
NW = NC * NS
KS = 128
NK = 4
CHUNK = NK * KS
ROWS_PER_W = B // NW // KS
NITER = ROWS_PER_W // NK


def _emb_body(w_hbm, idx_hbm, out_hbm, idx_v, rows_v, gsem):
    wid = lax.axis_index("s") * NC + lax.axis_index("c")
    row0 = wid * ROWS_PER_W

    def step(g, carry):
        r = row0 + g * NK
        pltpu.sync_copy(idx_hbm.at[pl.ds(r, NK)], idx_v)
        cps = [
            pltpu.async_copy(
                w_hbm.at[idx_v.at[j]],
                rows_v.at[pl.ds(j * KS, KS)],
                gsem,
            )
            for j in range(NK)
        ]
        for c in cps:
            c.wait()
        pltpu.sync_copy(rows_v, out_hbm.at[pl.ds(r * KS, CHUNK)])
        return carry

    lax.fori_loop(0, NITER, step, 0)


_emb = functools.partial(
    pl.kernel,
    out_type=jax.ShapeDtypeStruct((B, D), jnp.float32),
    mesh=plsc.VectorSubcoreMesh(core_axis_name="c", subcore_axis_name="s"),
    scratch_types=[
        pltpu.VMEM((NK, KS), jnp.int32),
        pltpu.VMEM((CHUNK, D), jnp.float32),
        pltpu.SemaphoreType.DMA,
    ],
    compiler_params=pltpu.CompilerParams(use_tc_tiling_on_sc=False),
)(_emb_body)


def kernel(joint_indices, weight):
    idx2d = joint_indices.reshape(-1, KS).astype(jnp.int32)
    out = _emb(weight, idx2d)
    return out.reshape(joint_indices.shape + (D,))


# speedup vs baseline: 4.1672x; 1.0046x over previous
import functools

import jax
import jax.numpy as jnp
from jax import lax
from jax.experimental import pallas as pl
from jax.experimental.pallas import tpu as pltpu
from jax.experimental.pallas import tpu_sc as plsc

NUM_ROWS = 1000
D = 64
B = 16384 * 200

NC, NS = 2, 16
NW = NC * NS
KS = 128
NK = 5
CHUNK = NK * KS
ROWS_PER_W = B // NW // KS
NITER = ROWS_PER_W // NK
NPAIR = (NITER - 2) // 2


def _emb_body(w_hbm, idx_hbm, out_hbm, ia, ib, ra, rb, isem, gsem, osem):
    wid = lax.axis_index("s") * NC + lax.axis_index("c")
    row0 = wid * ROWS_PER_W

    def idx_row(g):
        return row0 + lax.min(g, NITER - 1) * NK

    def issue_idx(g, dst):
        pltpu.async_copy(idx_hbm.at[pl.ds(idx_row(g), NK)], dst, isem)

    def wait_idx(dst):
        pltpu.make_async_copy(idx_hbm.at[pl.ds(row0, NK)], dst, isem).wait()

    def issue_gathers(src_idx, dst_rows, g):
        for j in range(NK):
            pltpu.async_copy(
                w_hbm.at[src_idx.at[j]],
                dst_rows.at[pl.ds(j * KS, KS)],
                gsem,
            )

    def wait_gathers(src_idx, dst_rows):
        for j in range(NK):
            pltpu.make_async_copy(
                w_hbm.at[src_idx.at[j]],
                dst_rows.at[pl.ds(j * KS, KS)],
                gsem,
            ).wait()

    def issue_write(g, src_rows):
        pltpu.async_copy(
            src_rows, out_hbm.at[pl.ds(idx_row(g) * KS, CHUNK)], osem
        )

    def wait_write(src_rows):
        pltpu.make_async_copy(
            src_rows, out_hbm.at[pl.ds(row0 * KS, CHUNK)], osem
        ).wait()

    def half(g, cur_i, cur_r, nxt_i, nxt_r, first):
        wait_gathers(cur_i, cur_r)
        issue_idx(g + 2, cur_i)
        issue_write(g, cur_r)
        if not first:
            wait_write(nxt_r)
        wait_idx(nxt_i)
        issue_gathers(nxt_i, nxt_r, g + 1)

    pltpu.sync_copy(idx_hbm.at[pl.ds(row0, NK)], ia)
    issue_gathers(ia, ra, 0)
    issue_idx(1, ib)

    half(0, ia, ra, ib, rb, first=True)

    def body(t, carry):
        g = 2 * t + 1
        half(g, ib, rb, ia, ra, first=False)
        half(g + 1, ia, ra, ib, rb, first=False)
        return carry

    lax.fori_loop(0, NPAIR, body, 0)

    half(NITER - 1, ib, rb, ia, ra, first=False)

    wait_gathers(ia, ra)
    wait_idx(ia)
    wait_write(rb)


_emb = functools.partial(
    pl.kernel,
    out_type=jax.ShapeDtypeStruct((B, D), jnp.float32),
    mesh=plsc.VectorSubcoreMesh(core_axis_name="c", subcore_axis_name="s"),
    scratch_types=[
        pltpu.VMEM((NK, KS), jnp.int32),
        pltpu.VMEM((NK, KS), jnp.int32),
        pltpu.VMEM((CHUNK, D), jnp.float32),
        pltpu.VMEM((CHUNK, D), jnp.float32),
        pltpu.SemaphoreType.DMA,
        pltpu.SemaphoreType.DMA,
        pltpu.SemaphoreType.DMA,
    ],
    compiler_params=pltpu.CompilerParams(use_tc_tiling_on_sc=False),
)(_emb_body)


def kernel(joint_indices, weight):
    idx2d = joint_indices.reshape(-1, KS).astype(jnp.int32)
    out = _emb(weight, idx2d)
    return out.reshape(joint_indices.shape + (D,))


# speedup vs baseline: 5.7770x; 1.3863x over previous
import functools

import jax
import jax.numpy as jnp
from jax import lax
from jax.experimental import pallas as pl
from jax.experimental.pallas import tpu as pltpu
from jax.experimental.pallas import tpu_sc as plsc

NUM_ROWS = 1000
D = 64
DP = 64
B = 16384 * 200

NC, NS = 2, 16
NW = NC * NS
KS = 128
NK = 2
CHUNK = NK * KS
PER_W = B // NW
NITER = PER_W // CHUNK
NPAIR = (NITER - 2) // 2


def _emb_body(w_hbm, idx_hbm, out_hbm, tbl, ia, ib, ra, rb, isem, gsem, osem):
    sid = lax.axis_index("s")
    wid = sid * NC + lax.axis_index("c")
    base = wid * PER_W

    @pl.when(sid == 0)
    def _():
        pltpu.sync_copy(w_hbm, tbl)

    plsc.subcore_barrier()

    def off(g):
        return base + lax.min(g, NITER - 1) * CHUNK

    def issue_idx(g, dst):
        pltpu.async_copy(idx_hbm.at[pl.ds(off(g), CHUNK)], dst, isem)

    def wait_idx(dst):
        pltpu.make_async_copy(idx_hbm.at[pl.ds(base, CHUNK)], dst, isem).wait()

    def issue_gathers(src_idx, dst_rows):
        for j in range(NK):
            pltpu.async_copy(
                tbl.at[src_idx.at[pl.ds(j * KS, KS)]],
                dst_rows.at[pl.ds(j * KS, KS)],
                gsem,
            )

    def wait_gathers(src_idx, dst_rows):
        for j in range(NK):
            pltpu.make_async_copy(
                tbl.at[src_idx.at[pl.ds(j * KS, KS)]],
                dst_rows.at[pl.ds(j * KS, KS)],
                gsem,
            ).wait()

    def issue_write(g, src_rows):
        pltpu.async_copy(src_rows, out_hbm.at[pl.ds(off(g), CHUNK)], osem)

    def wait_write(src_rows):
        pltpu.make_async_copy(
            src_rows, out_hbm.at[pl.ds(base, CHUNK)], osem
        ).wait()

    def half(g, cur_i, cur_r, nxt_i, nxt_r, first):
        wait_gathers(cur_i, cur_r)
        issue_idx(g + 2, cur_i)
        issue_write(g, cur_r)
        if not first:
            wait_write(nxt_r)
        wait_idx(nxt_i)
        issue_gathers(nxt_i, nxt_r)

    pltpu.sync_copy(idx_hbm.at[pl.ds(base, CHUNK)], ia)
    issue_gathers(ia, ra)
    issue_idx(1, ib)

    half(0, ia, ra, ib, rb, first=True)

    def body(t, carry):
        g = 2 * t + 1
        half(g, ib, rb, ia, ra, first=False)
        half(g + 1, ia, ra, ib, rb, first=False)
        return carry

    lax.fori_loop(0, NPAIR, body, 0)

    half(NITER - 1, ib, rb, ia, ra, first=False)

    wait_gathers(ia, ra)
    wait_idx(ia)
    wait_write(rb)


_emb = functools.partial(
    pl.kernel,
    out_type=jax.ShapeDtypeStruct((B, D), jnp.float32),
    mesh=plsc.VectorSubcoreMesh(core_axis_name="c", subcore_axis_name="s"),
    scratch_types=[
        pltpu.VMEM_SHARED((NUM_ROWS, DP), jnp.float32),
        pltpu.VMEM((CHUNK,), jnp.int32),
        pltpu.VMEM((CHUNK,), jnp.int32),
        pltpu.VMEM((CHUNK, D), jnp.float32),
        pltpu.VMEM((CHUNK, D), jnp.float32),
        pltpu.SemaphoreType.DMA,
        pltpu.SemaphoreType.DMA,
        pltpu.SemaphoreType.DMA,
    ],
    compiler_params=pltpu.CompilerParams(use_tc_tiling_on_sc=False),
)(_emb_body)


def kernel(joint_indices, weight):
    idx1d = joint_indices.reshape(-1).astype(jnp.int32)
    out = _emb(weight, idx1d)
    return out.reshape(joint_indices.shape + (D,))


# speedup vs baseline: 5.8163x; 1.0068x over previous
import functools

import jax
import jax.numpy as jnp
from jax import lax
from jax.experimental import pallas as pl
from jax.experimental.pallas import tpu as pltpu
from jax.experimental.pallas import tpu_sc as plsc

NUM_ROWS = 1000
D = 64
N0 = 16384
N1 = 200
SPLITS = (0, 104)
LENS = (104, 96)

NC, NS = 2, 16
NW = NC * NS
G = 4
PER_W = N0 // NW
NITER = PER_W // G
NPAIR = (NITER - 2) // 2


def _emb_body(w_hbm, idx_hbm, out_hbm, tbl, ia, ib, ra, rb, isem, gsem, osem):
    sid = lax.axis_index("s")
    wid = sid * NC + lax.axis_index("c")
    base = wid * PER_W

    @pl.when(sid == 0)
    def _():
        pltpu.sync_copy(w_hbm, tbl)

    plsc.subcore_barrier()

    def off(g):
        return base + lax.min(g, NITER - 1) * G

    def issue_idx(g, dst):
        pltpu.async_copy(idx_hbm.at[pl.ds(off(g), G)], dst, isem)

    def wait_idx(dst):
        pltpu.make_async_copy(idx_hbm.at[pl.ds(base, G)], dst, isem).wait()

    def issue_gathers(src_idx, dst_rows):
        for k in range(G):
            for s, n in zip(SPLITS, LENS):
                pltpu.async_copy(
                    tbl.at[src_idx.at[k, pl.ds(s, n)]],
                    dst_rows.at[k, pl.ds(s, n)],
                    gsem,
                )

    def wait_gathers(src_idx, dst_rows):
        for k in range(G):
            for s, n in zip(SPLITS, LENS):
                pltpu.make_async_copy(
                    tbl.at[src_idx.at[k, pl.ds(s, n)]],
                    dst_rows.at[k, pl.ds(s, n)],
                    gsem,
                ).wait()

    def issue_write(g, src_rows):
        pltpu.async_copy(src_rows, out_hbm.at[pl.ds(off(g), G)], osem)

    def wait_write(src_rows):
        pltpu.make_async_copy(
            src_rows, out_hbm.at[pl.ds(base, G)], osem
        ).wait()

    def half(g, cur_i, cur_r, nxt_i, nxt_r, first):
        wait_gathers(cur_i, cur_r)
        issue_idx(g + 2, cur_i)
        issue_write(g, cur_r)
        if not first:
            wait_write(nxt_r)
        wait_idx(nxt_i)
        issue_gathers(nxt_i, nxt_r)

    pltpu.sync_copy(idx_hbm.at[pl.ds(base, G)], ia)
    issue_gathers(ia, ra)
    issue_idx(1, ib)

    half(0, ia, ra, ib, rb, first=True)

    def body(t, carry):
        g = 2 * t + 1
        half(g, ib, rb, ia, ra, first=False)
        half(g + 1, ia, ra, ib, rb, first=False)
        return carry

    lax.fori_loop(0, NPAIR, body, 0)

    half(NITER - 1, ib, rb, ia, ra, first=False)

    wait_gathers(ia, ra)
    wait_idx(ia)
    wait_write(rb)


_emb = functools.partial(
    pl.kernel,
    out_type=jax.ShapeDtypeStruct((N0, N1, D), jnp.float32),
    mesh=plsc.VectorSubcoreMesh(core_axis_name="c", subcore_axis_name="s"),
    scratch_types=[
        pltpu.VMEM_SHARED((NUM_ROWS, D), jnp.float32),
        pltpu.VMEM((G, N1), jnp.int32),
        pltpu.VMEM((G, N1), jnp.int32),
        pltpu.VMEM((G, N1, D), jnp.float32),
        pltpu.VMEM((G, N1, D), jnp.float32),
        pltpu.SemaphoreType.DMA,
        pltpu.SemaphoreType.DMA,
        pltpu.SemaphoreType.DMA,
    ],
    compiler_params=pltpu.CompilerParams(use_tc_tiling_on_sc=False),
)(_emb_body)


def kernel(joint_indices, weight):
    return _emb(weight, joint_indices.astype(jnp.int32))
